# Initial kernel scaffold; baseline (speedup 1.0000x reference)
import jax
import jax.numpy as jnp
from jax.experimental import pallas as pl


def kernel(scalar_feats, coord_feats, positions, edge_index, params):
    raise NotImplementedError("write your pallas kernel here")


# speedup vs baseline: 10.4752x; 10.4752x over previous
import functools
import math

import jax
import jax.numpy as jnp
from jax.experimental import pallas as pl
from jax.experimental.pallas import tpu as pltpu

RBF_DIM = 16
RBF_DMAX = 15.0
NORM = 10.0


def _sigmoid(x):
    return 1.0 / (1.0 + jnp.exp(-x))


def _edge_kernel(g1, g2, w0, w1, w2, whc, whu, wu, b_out, wg, bg, out):
    S = 128
    V = 16
    g1v = g1[...]
    ps = g1v[:, S + 3 * V:S + 3 * V + 3]
    pd = g2[:, 0:3]
    xd = pd - ps
    d2 = jnp.sum(xd * xd, axis=1, keepdims=True)
    dist = jnp.sqrt(jnp.clip(d2, 1e-8))
    unit = xd / dist

    mu = (jnp.arange(RBF_DIM, dtype=jnp.int32).astype(jnp.float32)
          * (RBF_DMAX / (RBF_DIM - 1)))[None, :]
    sigma = RBF_DMAX / RBF_DIM
    rbf = jnp.exp(-(((dist - mu) / sigma) ** 2))

    whc_v = whc[...]
    whu_v = whu[...]
    wu_v = wu[...]
    vh = []
    for c in range(3):
        coord_c = g1v[:, S + V * c:S + V * (c + 1)]
        vh_c = jnp.dot(coord_c, whc_v, preferred_element_type=jnp.float32)
        vh_c = vh_c + unit[:, c:c + 1] * whu_v
        vh.append(vh_c)
    sh2 = vh[0] * vh[0] + vh[1] * vh[1] + vh[2] * vh[2]
    sh = jnp.sqrt(jnp.clip(sh2, 1e-8))

    lin = (jnp.dot(g1v[:, :S], w0[...], preferred_element_type=jnp.float32)
           + jnp.dot(rbf, w1[...], preferred_element_type=jnp.float32)
           + jnp.dot(sh, w2[...], preferred_element_type=jnp.float32)
           + b_out[...])
    feats = lin * _sigmoid(lin)
    gate = _sigmoid(jnp.dot(feats, wg[...], preferred_element_type=jnp.float32) + bg[...])

    out[:, :S] = feats
    for c in range(3):
        vu_c = jnp.dot(vh[c], wu_v, preferred_element_type=jnp.float32)
        out[:, S + V * c:S + V * (c + 1)] = gate * vu_c


def _node_kernel(agg0, agg1, sf, cf, w0, w1, whc, wu, b_out, wg, bg,
                 g_msg, b_msg, g_upd, b_upd, out_s, out_v):
    S = 128
    V = 16
    agg = (agg0[...] + agg1[...]) * (1.0 / NORM)
    agg_s = agg[:, :S]
    mu = jnp.mean(agg_s, axis=1, keepdims=True)
    var = jnp.mean((agg_s - mu) ** 2, axis=1, keepdims=True)
    nf = (agg_s - mu) / jnp.sqrt(var + 1e-5) * g_msg[...] + b_msg[...]
    av = [agg[:, S + V * c:S + V * (c + 1)] for c in range(3)]
    nu = jnp.clip(av[0] * av[0] + av[1] * av[1] + av[2] * av[2], 1e-8)
    vn = jnp.sqrt(jnp.mean(nu, axis=1, keepdims=True))

    s1 = sf[...] + nf
    cfv = cf[...]
    v1 = [cfv[:, V * c:V * (c + 1)] + av[c] / vn for c in range(3)]

    whc_v = whc[...]
    wu_v = wu[...]
    vh = [jnp.dot(v1[c], whc_v, preferred_element_type=jnp.float32) for c in range(3)]
    sh = jnp.sqrt(jnp.clip(vh[0] * vh[0] + vh[1] * vh[1] + vh[2] * vh[2], 1e-8))
    lin = (jnp.dot(s1, w0[...], preferred_element_type=jnp.float32)
           + jnp.dot(sh, w1[...], preferred_element_type=jnp.float32)
           + b_out[...])
    feats = lin * _sigmoid(lin)
    gate = _sigmoid(jnp.dot(feats, wg[...], preferred_element_type=jnp.float32) + bg[...])
    uv = [gate * jnp.dot(vh[c], wu_v, preferred_element_type=jnp.float32) for c in range(3)]

    s2 = s1 + feats
    v2 = [v1[c] + uv[c] for c in range(3)]
    mu2 = jnp.mean(s2, axis=1, keepdims=True)
    var2 = jnp.mean((s2 - mu2) ** 2, axis=1, keepdims=True)
    out_s[...] = (s2 - mu2) / jnp.sqrt(var2 + 1e-5) * g_upd[...] + b_upd[...]
    nu2 = jnp.clip(v2[0] * v2[0] + v2[1] * v2[1] + v2[2] * v2[2], 1e-8)
    vn2 = jnp.sqrt(jnp.mean(nu2, axis=1, keepdims=True))
    for c in range(3):
        out_v[:, V * c:V * (c + 1)] = v2[c] / vn2


def kernel(scalar_feats, coord_feats, positions, edge_index, params):
    N, S = scalar_feats.shape
    V = coord_feats.shape[1]
    E = edge_index.shape[1]
    src = edge_index[0]
    dst = edge_index[1]

    coord_cm = coord_feats.transpose(0, 2, 1).reshape(N, 3 * V)
    tbl = jnp.concatenate(
        [scalar_feats, coord_cm, positions,
         jnp.zeros((N, 13), jnp.float32)], axis=1)
    posp = jnp.concatenate(
        [positions, jnp.zeros((N, 13), jnp.float32)], axis=1)

    g1 = tbl[src]
    g2 = posp[dst]

    pm = params['msg']
    w_out = pm['W_out']
    TB = 2000
    msg = pl.pallas_call(
        _edge_kernel,
        grid=(E // TB,),
        in_specs=[
            pl.BlockSpec((TB, 192), lambda i: (i, 0)),
            pl.BlockSpec((TB, 16), lambda i: (i, 0)),
        ] + [pl.BlockSpec(s, lambda i: (0, 0)) for s in
             [(S, S), (RBF_DIM, S), (V + 1, S), (V, V + 1), (1, V + 1),
              (V + 1, V), (1, S), (S, V), (1, V)]],
        out_specs=pl.BlockSpec((TB, S + 3 * V), lambda i: (i, 0)),
        out_shape=jax.ShapeDtypeStruct((E, S + 3 * V), jnp.float32),
    )(g1, g2,
      w_out[:S], w_out[S:S + RBF_DIM], w_out[S + RBF_DIM:],
      pm['Wh'][:V], pm['Wh'][V:V + 1], pm['Wu'], pm['b_out'][None, :],
      pm['W_gate'], pm['b_gate'][None, :])

    agg0 = jax.ops.segment_sum(msg, dst, num_segments=N)
    agg1 = jnp.zeros_like(agg0)

    pu = params['upd']
    wu_out = pu['W_out']
    NB = 2000
    out_s, out_v = pl.pallas_call(
        _node_kernel,
        grid=(N // NB,),
        in_specs=[
            pl.BlockSpec((NB, S + 3 * V), lambda i: (i, 0)),
            pl.BlockSpec((NB, S + 3 * V), lambda i: (i, 0)),
            pl.BlockSpec((NB, S), lambda i: (i, 0)),
            pl.BlockSpec((NB, 3 * V), lambda i: (i, 0)),
        ] + [pl.BlockSpec(s, lambda i: (0, 0)) for s in
             [(S, S), (V, S), (V, V), (V, V), (1, S), (S, V), (1, V),
              (1, S), (1, S), (1, S), (1, S)]],
        out_specs=[
            pl.BlockSpec((NB, S), lambda i: (i, 0)),
            pl.BlockSpec((NB, 3 * V), lambda i: (i, 0)),
        ],
        out_shape=[
            jax.ShapeDtypeStruct((N, S), jnp.float32),
            jax.ShapeDtypeStruct((N, 3 * V), jnp.float32),
        ],
    )(agg0, agg1, scalar_feats, coord_cm,
      wu_out[:S], wu_out[S:], pu['Wh'], pu['Wu'], pu['b_out'][None, :],
      pu['W_gate'], pu['b_gate'][None, :],
      params['msg_ln']['gamma'][None, :], params['msg_ln']['beta'][None, :],
      params['upd_ln']['gamma'][None, :], params['upd_ln']['beta'][None, :])

    v2 = out_v.reshape(N, 3, V).transpose(0, 2, 1)
    return out_s, v2


# speedup vs baseline: 15.9880x; 1.5263x over previous
import functools
import math

import jax
import jax.numpy as jnp
from jax import lax
from jax.experimental import pallas as pl
from jax.experimental.pallas import tpu as pltpu
from jax.experimental.pallas import tpu_sc as plsc

RBF_DIM = 16
RBF_DMAX = 15.0
NORM = 10.0


def _sigmoid(x):
    return 1.0 / (1.0 + jnp.exp(-x))


def _edge_kernel(g1, g2, w0, w1, w2, whc, whu, wu, b_out, wg, bg, out):
    S = 128
    V = 16
    g1v = g1[...]
    ps = g1v[:, S + 3 * V:S + 3 * V + 3]
    pd = g2[:, 0:3]
    xd = pd - ps
    d2 = jnp.sum(xd * xd, axis=1, keepdims=True)
    dist = jnp.sqrt(jnp.clip(d2, 1e-8))
    unit = xd / dist

    mu = (jnp.arange(RBF_DIM, dtype=jnp.int32).astype(jnp.float32)
          * (RBF_DMAX / (RBF_DIM - 1)))[None, :]
    sigma = RBF_DMAX / RBF_DIM
    rbf = jnp.exp(-(((dist - mu) / sigma) ** 2))

    whc_v = whc[...]
    whu_v = whu[...]
    wu_v = wu[...]
    vh = []
    for c in range(3):
        coord_c = g1v[:, S + V * c:S + V * (c + 1)]
        vh_c = jnp.dot(coord_c, whc_v, preferred_element_type=jnp.float32)
        vh_c = vh_c + unit[:, c:c + 1] * whu_v
        vh.append(vh_c)
    sh2 = vh[0] * vh[0] + vh[1] * vh[1] + vh[2] * vh[2]
    sh = jnp.sqrt(jnp.clip(sh2, 1e-8))

    lin = (jnp.dot(g1v[:, :S], w0[...], preferred_element_type=jnp.float32)
           + jnp.dot(rbf, w1[...], preferred_element_type=jnp.float32)
           + jnp.dot(sh, w2[...], preferred_element_type=jnp.float32)
           + b_out[...])
    feats = lin * _sigmoid(lin)
    gate = _sigmoid(jnp.dot(feats, wg[...], preferred_element_type=jnp.float32) + bg[...])

    out[:, :S] = feats
    for c in range(3):
        vu_c = jnp.dot(vh[c], wu_v, preferred_element_type=jnp.float32)
        out[:, S + V * c:S + V * (c + 1)] = gate * vu_c


def _node_kernel(agg0, agg1, sf, cf, w0, w1, whc, wu, b_out, wg, bg,
                 g_msg, b_msg, g_upd, b_upd, out_s, out_v):
    S = 128
    V = 16
    agg = (agg0[...] + agg1[...]) * (1.0 / NORM)
    agg_s = agg[:, :S]
    mu = jnp.mean(agg_s, axis=1, keepdims=True)
    var = jnp.mean((agg_s - mu) ** 2, axis=1, keepdims=True)
    nf = (agg_s - mu) / jnp.sqrt(var + 1e-5) * g_msg[...] + b_msg[...]
    av = [agg[:, S + V * c:S + V * (c + 1)] for c in range(3)]
    nu = jnp.clip(av[0] * av[0] + av[1] * av[1] + av[2] * av[2], 1e-8)
    vn = jnp.sqrt(jnp.mean(nu, axis=1, keepdims=True))

    s1 = sf[...] + nf
    cfv = cf[...]
    v1 = [cfv[:, V * c:V * (c + 1)] + av[c] / vn for c in range(3)]

    whc_v = whc[...]
    wu_v = wu[...]
    vh = [jnp.dot(v1[c], whc_v, preferred_element_type=jnp.float32) for c in range(3)]
    sh = jnp.sqrt(jnp.clip(vh[0] * vh[0] + vh[1] * vh[1] + vh[2] * vh[2], 1e-8))
    lin = (jnp.dot(s1, w0[...], preferred_element_type=jnp.float32)
           + jnp.dot(sh, w1[...], preferred_element_type=jnp.float32)
           + b_out[...])
    feats = lin * _sigmoid(lin)
    gate = _sigmoid(jnp.dot(feats, wg[...], preferred_element_type=jnp.float32) + bg[...])
    uv = [gate * jnp.dot(vh[c], wu_v, preferred_element_type=jnp.float32) for c in range(3)]

    s2 = s1 + feats
    v2 = [v1[c] + uv[c] for c in range(3)]
    mu2 = jnp.mean(s2, axis=1, keepdims=True)
    var2 = jnp.mean((s2 - mu2) ** 2, axis=1, keepdims=True)
    out_s[...] = (s2 - mu2) / jnp.sqrt(var2 + 1e-5) * g_upd[...] + b_upd[...]
    nu2 = jnp.clip(v2[0] * v2[0] + v2[1] * v2[1] + v2[2] * v2[2], 1e-8)
    vn2 = jnp.sqrt(jnp.mean(nu2, axis=1, keepdims=True))
    for c in range(3):
        out_v[:, V * c:V * (c + 1)] = v2[c] / vn2


def _sc_gather(tbl, posp, src, dst):
    N = tbl.shape[0]
    E = src.shape[0]
    D1 = tbl.shape[1]
    D2 = posp.shape[1]
    NW = 32
    EPW = E // NW
    CH = 128
    KFULL = EPW // CH
    TAIL = EPW - KFULL * CH

    mesh = plsc.VectorSubcoreMesh(core_axis_name="c", subcore_axis_name="s")

    @functools.partial(
        pl.kernel, mesh=mesh,
        compiler_params=pltpu.CompilerParams(use_tc_tiling_on_sc=False),
        out_type=[
            jax.ShapeDtypeStruct((E, D1), jnp.float32),
            jax.ShapeDtypeStruct((E, D2), jnp.float32),
        ],
        scratch_types=[
            pltpu.VMEM((EPW,), jnp.int32),
            pltpu.VMEM((EPW,), jnp.int32),
            pltpu.VMEM((2, CH, D1), jnp.float32),
            pltpu.VMEM((2, CH, D2), jnp.float32),
            pltpu.VMEM((TAIL, D1), jnp.float32),
            pltpu.VMEM((TAIL, D2), jnp.float32),
            pltpu.SemaphoreType.DMA,
            pltpu.SemaphoreType.DMA,
            pltpu.SemaphoreType.DMA,
            pltpu.SemaphoreType.DMA,
        ],
    )
    def gk(tbl_h, posp_h, src_h, dst_h, g1_h, g2_h,
           idxs_v, idxd_v, rows_v, prow_v, trow_v, tprow_v,
           sg0, sg1, sp0, sp1):
        wid = lax.axis_index("s") * 2 + lax.axis_index("c")
        base = pl.multiple_of(wid * EPW, 8)
        pltpu.sync_copy(src_h.at[pl.ds(base, EPW)], idxs_v)
        pltpu.sync_copy(dst_h.at[pl.ds(base, EPW)], idxd_v)
        sgs = [sg0, sg1]
        sps = [sp0, sp1]

        def body(i, carry):
            cps = []
            for b in range(2):
                k = i * 2 + b
                o = pl.multiple_of(k * CH, 8)
                cps.append(pltpu.async_copy(
                    tbl_h.at[idxs_v.at[pl.ds(o, CH)]], rows_v.at[b], sgs[b]))
                cps.append(pltpu.async_copy(
                    posp_h.at[idxd_v.at[pl.ds(o, CH)]], prow_v.at[b], sps[b]))
            for b in range(2):
                k = i * 2 + b
                oo = pl.multiple_of(base + k * CH, 8)
                cps[2 * b].wait()
                pltpu.sync_copy(rows_v.at[b], g1_h.at[pl.ds(oo, CH)])
                cps[2 * b + 1].wait()
                pltpu.sync_copy(prow_v.at[b], g2_h.at[pl.ds(oo, CH)])
            return carry

        lax.fori_loop(0, KFULL // 2, body, 0, unroll=False)

        ot = pl.multiple_of(KFULL * CH, 8)
        oot = pl.multiple_of(base + KFULL * CH, 8)
        pltpu.async_copy(tbl_h.at[idxs_v.at[pl.ds(ot, TAIL)]], trow_v, sg0).wait()
        pltpu.sync_copy(trow_v, g1_h.at[pl.ds(oot, TAIL)])
        pltpu.async_copy(posp_h.at[idxd_v.at[pl.ds(ot, TAIL)]], tprow_v, sp0).wait()
        pltpu.sync_copy(tprow_v, g2_h.at[pl.ds(oot, TAIL)])

    return gk(tbl, posp, src, dst)


def kernel(scalar_feats, coord_feats, positions, edge_index, params):
    N, S = scalar_feats.shape
    V = coord_feats.shape[1]
    E = edge_index.shape[1]
    src = edge_index[0]
    dst = edge_index[1]

    coord_cm = coord_feats.transpose(0, 2, 1).reshape(N, 3 * V)
    tbl = jnp.concatenate(
        [scalar_feats, coord_cm, positions,
         jnp.zeros((N, 13), jnp.float32)], axis=1)
    posp = jnp.concatenate(
        [positions, jnp.zeros((N, 13), jnp.float32)], axis=1)

    g1, g2 = _sc_gather(tbl, posp, src, dst)

    pm = params['msg']
    w_out = pm['W_out']
    TB = 2000
    msg = pl.pallas_call(
        _edge_kernel,
        grid=(E // TB,),
        in_specs=[
            pl.BlockSpec((TB, 192), lambda i: (i, 0)),
            pl.BlockSpec((TB, 16), lambda i: (i, 0)),
        ] + [pl.BlockSpec(s, lambda i: (0, 0)) for s in
             [(S, S), (RBF_DIM, S), (V + 1, S), (V, V + 1), (1, V + 1),
              (V + 1, V), (1, S), (S, V), (1, V)]],
        out_specs=pl.BlockSpec((TB, S + 3 * V), lambda i: (i, 0)),
        out_shape=jax.ShapeDtypeStruct((E, S + 3 * V), jnp.float32),
    )(g1, g2,
      w_out[:S], w_out[S:S + RBF_DIM], w_out[S + RBF_DIM:],
      pm['Wh'][:V], pm['Wh'][V:V + 1], pm['Wu'], pm['b_out'][None, :],
      pm['W_gate'], pm['b_gate'][None, :])

    agg0 = jax.ops.segment_sum(msg, dst, num_segments=N)
    agg1 = jnp.zeros_like(agg0)

    pu = params['upd']
    wu_out = pu['W_out']
    NB = 2000
    out_s, out_v = pl.pallas_call(
        _node_kernel,
        grid=(N // NB,),
        in_specs=[
            pl.BlockSpec((NB, S + 3 * V), lambda i: (i, 0)),
            pl.BlockSpec((NB, S + 3 * V), lambda i: (i, 0)),
            pl.BlockSpec((NB, S), lambda i: (i, 0)),
            pl.BlockSpec((NB, 3 * V), lambda i: (i, 0)),
        ] + [pl.BlockSpec(s, lambda i: (0, 0)) for s in
             [(S, S), (V, S), (V, V), (V, V), (1, S), (S, V), (1, V),
              (1, S), (1, S), (1, S), (1, S)]],
        out_specs=[
            pl.BlockSpec((NB, S), lambda i: (i, 0)),
            pl.BlockSpec((NB, 3 * V), lambda i: (i, 0)),
        ],
        out_shape=[
            jax.ShapeDtypeStruct((N, S), jnp.float32),
            jax.ShapeDtypeStruct((N, 3 * V), jnp.float32),
        ],
    )(agg0, agg1, scalar_feats, coord_cm,
      wu_out[:S], wu_out[S:], pu['Wh'], pu['Wu'], pu['b_out'][None, :],
      pu['W_gate'], pu['b_gate'][None, :],
      params['msg_ln']['gamma'][None, :], params['msg_ln']['beta'][None, :],
      params['upd_ln']['gamma'][None, :], params['upd_ln']['beta'][None, :])

    v2 = out_v.reshape(N, 3, V).transpose(0, 2, 1)
    return out_s, v2


# speedup vs baseline: 19.3824x; 1.2123x over previous
import functools
import math

import jax
import jax.numpy as jnp
from jax import lax
from jax.experimental import pallas as pl
from jax.experimental.pallas import tpu as pltpu
from jax.experimental.pallas import tpu_sc as plsc

RBF_DIM = 16
RBF_DMAX = 15.0
NORM = 10.0


def _sigmoid(x):
    return 1.0 / (1.0 + jnp.exp(-x))


def _edge_kernel(g1, g2, w0, w1, w2, whc, whu, wu, b_out, wg, bg, out):
    S = 128
    V = 16
    g1v = g1[...]
    ps = g1v[:, S + 3 * V:S + 3 * V + 3]
    pd = g2[:, 0:3]
    xd = pd - ps
    d2 = jnp.sum(xd * xd, axis=1, keepdims=True)
    dist = jnp.sqrt(jnp.clip(d2, 1e-8))
    unit = xd / dist

    mu = (jnp.arange(RBF_DIM, dtype=jnp.int32).astype(jnp.float32)
          * (RBF_DMAX / (RBF_DIM - 1)))[None, :]
    sigma = RBF_DMAX / RBF_DIM
    rbf = jnp.exp(-(((dist - mu) / sigma) ** 2))

    whc_v = whc[...]
    whu_v = whu[...]
    wu_v = wu[...]
    vh = []
    for c in range(3):
        coord_c = g1v[:, S + V * c:S + V * (c + 1)]
        vh_c = jnp.dot(coord_c, whc_v, preferred_element_type=jnp.float32)
        vh_c = vh_c + unit[:, c:c + 1] * whu_v
        vh.append(vh_c)
    sh2 = vh[0] * vh[0] + vh[1] * vh[1] + vh[2] * vh[2]
    sh = jnp.sqrt(jnp.clip(sh2, 1e-8))

    lin = (jnp.dot(g1v[:, :S], w0[...], preferred_element_type=jnp.float32)
           + jnp.dot(rbf, w1[...], preferred_element_type=jnp.float32)
           + jnp.dot(sh, w2[...], preferred_element_type=jnp.float32)
           + b_out[...])
    feats = lin * _sigmoid(lin)
    gate = _sigmoid(jnp.dot(feats, wg[...], preferred_element_type=jnp.float32) + bg[...])

    out[:, :S] = feats
    for c in range(3):
        vu_c = jnp.dot(vh[c], wu_v, preferred_element_type=jnp.float32)
        out[:, S + V * c:S + V * (c + 1)] = gate * vu_c


def _node_kernel(agg0, agg1, sf, cf, w0, w1, whc, wu, b_out, wg, bg,
                 g_msg, b_msg, g_upd, b_upd, out_s, out_v):
    S = 128
    V = 16
    agg = (agg0[...] + agg1[...]) * (1.0 / NORM)
    agg_s = agg[:, :S]
    mu = jnp.mean(agg_s, axis=1, keepdims=True)
    var = jnp.mean((agg_s - mu) ** 2, axis=1, keepdims=True)
    nf = (agg_s - mu) / jnp.sqrt(var + 1e-5) * g_msg[...] + b_msg[...]
    av = [agg[:, S + V * c:S + V * (c + 1)] for c in range(3)]
    nu = jnp.clip(av[0] * av[0] + av[1] * av[1] + av[2] * av[2], 1e-8)
    vn = jnp.sqrt(jnp.mean(nu, axis=1, keepdims=True))

    s1 = sf[...] + nf
    cfv = cf[...]
    v1 = [cfv[:, V * c:V * (c + 1)] + av[c] / vn for c in range(3)]

    whc_v = whc[...]
    wu_v = wu[...]
    vh = [jnp.dot(v1[c], whc_v, preferred_element_type=jnp.float32) for c in range(3)]
    sh = jnp.sqrt(jnp.clip(vh[0] * vh[0] + vh[1] * vh[1] + vh[2] * vh[2], 1e-8))
    lin = (jnp.dot(s1, w0[...], preferred_element_type=jnp.float32)
           + jnp.dot(sh, w1[...], preferred_element_type=jnp.float32)
           + b_out[...])
    feats = lin * _sigmoid(lin)
    gate = _sigmoid(jnp.dot(feats, wg[...], preferred_element_type=jnp.float32) + bg[...])
    uv = [gate * jnp.dot(vh[c], wu_v, preferred_element_type=jnp.float32) for c in range(3)]

    s2 = s1 + feats
    v2 = [v1[c] + uv[c] for c in range(3)]
    mu2 = jnp.mean(s2, axis=1, keepdims=True)
    var2 = jnp.mean((s2 - mu2) ** 2, axis=1, keepdims=True)
    out_s[...] = (s2 - mu2) / jnp.sqrt(var2 + 1e-5) * g_upd[...] + b_upd[...]
    nu2 = jnp.clip(v2[0] * v2[0] + v2[1] * v2[1] + v2[2] * v2[2], 1e-8)
    vn2 = jnp.sqrt(jnp.mean(nu2, axis=1, keepdims=True))
    for c in range(3):
        out_v[:, V * c:V * (c + 1)] = v2[c] / vn2


def _sc_gather(tbl, posp, src, dst):
    N = tbl.shape[0]
    E = src.shape[0]
    D1 = tbl.shape[1]
    D2 = posp.shape[1]
    NW = 32
    EPW = E // NW
    CH = 128
    KFULL = EPW // CH
    TAIL = EPW - KFULL * CH

    mesh = plsc.VectorSubcoreMesh(core_axis_name="c", subcore_axis_name="s")

    @functools.partial(
        pl.kernel, mesh=mesh,
        compiler_params=pltpu.CompilerParams(use_tc_tiling_on_sc=False),
        out_type=[
            jax.ShapeDtypeStruct((E, D1), jnp.float32),
            jax.ShapeDtypeStruct((E, D2), jnp.float32),
        ],
        scratch_types=[
            pltpu.VMEM((EPW,), jnp.int32),
            pltpu.VMEM((EPW,), jnp.int32),
            pltpu.VMEM((2, CH, D1), jnp.float32),
            pltpu.VMEM((2, CH, D2), jnp.float32),
            pltpu.VMEM((TAIL, D1), jnp.float32),
            pltpu.VMEM((TAIL, D2), jnp.float32),
            pltpu.SemaphoreType.DMA,
            pltpu.SemaphoreType.DMA,
            pltpu.SemaphoreType.DMA,
            pltpu.SemaphoreType.DMA,
        ],
    )
    def gk(tbl_h, posp_h, src_h, dst_h, g1_h, g2_h,
           idxs_v, idxd_v, rows_v, prow_v, trow_v, tprow_v,
           sg0, sg1, sp0, sp1):
        wid = lax.axis_index("s") * 2 + lax.axis_index("c")
        base = pl.multiple_of(wid * EPW, 8)
        pltpu.sync_copy(src_h.at[pl.ds(base, EPW)], idxs_v)
        pltpu.sync_copy(dst_h.at[pl.ds(base, EPW)], idxd_v)
        sgs = [sg0, sg1]
        sps = [sp0, sp1]

        def body(i, carry):
            cps = []
            for b in range(2):
                k = i * 2 + b
                o = pl.multiple_of(k * CH, 8)
                cps.append(pltpu.async_copy(
                    tbl_h.at[idxs_v.at[pl.ds(o, CH)]], rows_v.at[b], sgs[b]))
                cps.append(pltpu.async_copy(
                    posp_h.at[idxd_v.at[pl.ds(o, CH)]], prow_v.at[b], sps[b]))
            for b in range(2):
                k = i * 2 + b
                oo = pl.multiple_of(base + k * CH, 8)
                cps[2 * b].wait()
                pltpu.sync_copy(rows_v.at[b], g1_h.at[pl.ds(oo, CH)])
                cps[2 * b + 1].wait()
                pltpu.sync_copy(prow_v.at[b], g2_h.at[pl.ds(oo, CH)])
            return carry

        lax.fori_loop(0, KFULL // 2, body, 0, unroll=False)

        ot = pl.multiple_of(KFULL * CH, 8)
        oot = pl.multiple_of(base + KFULL * CH, 8)
        pltpu.async_copy(tbl_h.at[idxs_v.at[pl.ds(ot, TAIL)]], trow_v, sg0).wait()
        pltpu.sync_copy(trow_v, g1_h.at[pl.ds(oot, TAIL)])
        pltpu.async_copy(posp_h.at[idxd_v.at[pl.ds(ot, TAIL)]], tprow_v, sp0).wait()
        pltpu.sync_copy(tprow_v, g2_h.at[pl.ds(oot, TAIL)])

    return gk(tbl, posp, src, dst)


def _sc_scatter(msg, dst, zinit):
    E, D = msg.shape
    NP = zinit.shape[0]
    NW = 32
    EPW = E // NW
    CH = 40
    KFULL = EPW // CH
    RPT = NP // 16

    mesh = plsc.VectorSubcoreMesh(core_axis_name="c", subcore_axis_name="s")

    @functools.partial(
        pl.kernel, mesh=mesh,
        compiler_params=pltpu.CompilerParams(use_tc_tiling_on_sc=False),
        out_type=jax.ShapeDtypeStruct((2, NP, D), jnp.float32),
        scratch_types=[
            pltpu.VMEM_SHARED((NP, D), jnp.float32),
            pltpu.VMEM((2, CH), jnp.int32),
            pltpu.VMEM((2, CH, D), jnp.float32),
            pltpu.SemaphoreType.DMA,
            pltpu.SemaphoreType.DMA,
        ],
    )
    def sk(msg_h, dst_h, zin_h, out_h,
           acc_s, idx_v, rows_v, s0, s1):
        cid = lax.axis_index("c")
        sid = lax.axis_index("s")
        wid = sid * 2 + cid
        base = pl.multiple_of(wid * EPW, 8)
        r0 = pl.multiple_of(sid * RPT, 8)

        pltpu.sync_copy(zin_h.at[pl.ds(r0, RPT)], acc_s.at[pl.ds(r0, RPT)])
        plsc.subcore_barrier()

        sems = [s0, s1]

        def body(i, carry):
            cps = []
            for b in range(2):
                k = i * 2 + b
                o = pl.multiple_of(base + k * CH, 8)
                cps.append(pltpu.async_copy(
                    dst_h.at[pl.ds(o, CH)], idx_v.at[b], sems[b]))
                cps.append(pltpu.async_copy(
                    msg_h.at[pl.ds(o, CH)], rows_v.at[b], sems[b]))
            for b in range(2):
                cps[2 * b].wait()
                cps[2 * b + 1].wait()
                pltpu.sync_copy(rows_v.at[b], acc_s.at[idx_v.at[b]], add=True)
            return carry

        lax.fori_loop(0, KFULL // 2, body, 0, unroll=False)

        plsc.subcore_barrier()
        pltpu.sync_copy(acc_s.at[pl.ds(r0, RPT)], out_h.at[cid, pl.ds(r0, RPT)])

    return sk(msg, dst, zinit)


def kernel(scalar_feats, coord_feats, positions, edge_index, params):
    N, S = scalar_feats.shape
    V = coord_feats.shape[1]
    E = edge_index.shape[1]
    src = edge_index[0]
    dst = edge_index[1]

    coord_cm = coord_feats.transpose(0, 2, 1).reshape(N, 3 * V)
    tbl = jnp.concatenate(
        [scalar_feats, coord_cm, positions,
         jnp.zeros((N, 13), jnp.float32)], axis=1)
    posp = jnp.concatenate(
        [positions, jnp.zeros((N, 13), jnp.float32)], axis=1)

    g1, g2 = _sc_gather(tbl, posp, src, dst)

    pm = params['msg']
    w_out = pm['W_out']
    TB = 2000
    msg = pl.pallas_call(
        _edge_kernel,
        grid=(E // TB,),
        in_specs=[
            pl.BlockSpec((TB, 192), lambda i: (i, 0)),
            pl.BlockSpec((TB, 16), lambda i: (i, 0)),
        ] + [pl.BlockSpec(s, lambda i: (0, 0)) for s in
             [(S, S), (RBF_DIM, S), (V + 1, S), (V, V + 1), (1, V + 1),
              (V + 1, V), (1, S), (S, V), (1, V)]],
        out_specs=pl.BlockSpec((TB, S + 3 * V), lambda i: (i, 0)),
        out_shape=jax.ShapeDtypeStruct((E, S + 3 * V), jnp.float32),
    )(g1, g2,
      w_out[:S], w_out[S:S + RBF_DIM], w_out[S + RBF_DIM:],
      pm['Wh'][:V], pm['Wh'][V:V + 1], pm['Wu'], pm['b_out'][None, :],
      pm['W_gate'], pm['b_gate'][None, :])

    NP = 16 * 632
    zinit = jnp.zeros((NP, S + 3 * V), jnp.float32)
    parts = _sc_scatter(msg, dst, zinit)
    agg0 = parts[0, :N]
    agg1 = parts[1, :N]

    pu = params['upd']
    wu_out = pu['W_out']
    NB = 2000
    out_s, out_v = pl.pallas_call(
        _node_kernel,
        grid=(N // NB,),
        in_specs=[
            pl.BlockSpec((NB, S + 3 * V), lambda i: (i, 0)),
            pl.BlockSpec((NB, S + 3 * V), lambda i: (i, 0)),
            pl.BlockSpec((NB, S), lambda i: (i, 0)),
            pl.BlockSpec((NB, 3 * V), lambda i: (i, 0)),
        ] + [pl.BlockSpec(s, lambda i: (0, 0)) for s in
             [(S, S), (V, S), (V, V), (V, V), (1, S), (S, V), (1, V),
              (1, S), (1, S), (1, S), (1, S)]],
        out_specs=[
            pl.BlockSpec((NB, S), lambda i: (i, 0)),
            pl.BlockSpec((NB, 3 * V), lambda i: (i, 0)),
        ],
        out_shape=[
            jax.ShapeDtypeStruct((N, S), jnp.float32),
            jax.ShapeDtypeStruct((N, 3 * V), jnp.float32),
        ],
    )(agg0, agg1, scalar_feats, coord_cm,
      wu_out[:S], wu_out[S:], pu['Wh'], pu['Wu'], pu['b_out'][None, :],
      pu['W_gate'], pu['b_gate'][None, :],
      params['msg_ln']['gamma'][None, :], params['msg_ln']['beta'][None, :],
      params['upd_ln']['gamma'][None, :], params['upd_ln']['beta'][None, :])

    v2 = out_v.reshape(N, 3, V).transpose(0, 2, 1)
    return out_s, v2
